# baseline (speedup 1.0000x reference)
import functools

import jax
import jax.numpy as jnp
from jax import lax
from jax.experimental import pallas as pl
from jax.experimental.pallas import tpu as pltpu

_EPS = 1e-3
_HI = lax.Precision.HIGHEST


def _coeffs(g, be, s, q, inv_m):
    mean = s * inv_m
    var = q * inv_m - mean * mean
    a = g * lax.rsqrt(var + _EPS)
    return a, be - mean * a


def _conv3x3(hb, w2b, H, W):
    hw = H * W
    cint = w2b.shape[1]
    idx = lax.broadcasted_iota(jnp.int32, (1, hw), 1)
    row = idx // W
    col = idx % W

    dxm = jnp.where(col >= 1, pltpu.roll(hb, shift=1, axis=1), 0)
    dxp = jnp.where(col <= W - 2, pltpu.roll(hb, shift=hw - 1, axis=1), 0)
    stack = jnp.concatenate([dxm, hb, dxp], axis=0)

    def wrow(dy):
        k = (dy + 1) * 3
        return jnp.concatenate([w2b[k], w2b[k + 1], w2b[k + 2]], axis=1)

    acc = jnp.dot(wrow(0), stack, preferred_element_type=jnp.float32)
    up = jnp.dot(wrow(-1), stack, preferred_element_type=jnp.float32)
    acc += jnp.where(row >= 1, pltpu.roll(up, shift=W, axis=1), 0.0)
    dn = jnp.dot(wrow(1), stack, preferred_element_type=jnp.float32)
    acc += jnp.where(row <= H - 2, pltpu.roll(dn, shift=hw - W, axis=1), 0.0)
    return acc


def _stage12_kernel(x_ref, w1_ref, w2_ref, b2_ref, g1_ref, be1_ref,
                    y2_ref, s2_ref, q2_ref,
                    xb_ref, m_ref, sx_ref, a1_ref, c1_ref, *, N, H, W):
    i = pl.program_id(0)

    @pl.when(i < N)
    def _phase_a():
        x = x_ref[0]
        xb = x.astype(jnp.bfloat16)
        xb_ref[i] = xb

        @pl.when(i == 0)
        def _():
            m_ref[...] = jnp.zeros_like(m_ref)
            sx_ref[...] = jnp.zeros_like(sx_ref)

        m_ref[...] += lax.dot_general(xb, xb, (((1,), (1,)), ((), ())),
                                      preferred_element_type=jnp.float32)
        sx_ref[...] += jnp.sum(x, axis=1, keepdims=True)

    @pl.when(i == N)
    def _coef1():
        w1 = w1_ref[...]
        s1 = jnp.dot(w1, sx_ref[...], precision=_HI,
                     preferred_element_type=jnp.float32)
        a = jnp.dot(w1, m_ref[...], precision=_HI,
                    preferred_element_type=jnp.float32)
        q1 = jnp.sum(a * w1, axis=1, keepdims=True)
        a1, c1 = _coeffs(g1_ref[...], be1_ref[...], s1, q1, 1.0 / (N * H * W))
        a1_ref[...] = a1
        c1_ref[...] = c1

    @pl.when(i >= N)
    def _phase_b():
        j = i - N
        w1b = w1_ref[...].astype(jnp.bfloat16)
        y1 = jnp.dot(w1b, xb_ref[j], preferred_element_type=jnp.float32)
        h = jnp.maximum(a1_ref[...] * y1 + c1_ref[...], 0.0)
        w2b = w2_ref[...].astype(jnp.bfloat16)
        y2 = _conv3x3(h.astype(jnp.bfloat16), w2b, H, W) + b2_ref[...]
        y2_ref[0] = y2.astype(jnp.bfloat16)

        @pl.when(i == N)
        def _():
            s2_ref[...] = jnp.zeros_like(s2_ref)
            q2_ref[...] = jnp.zeros_like(q2_ref)

        s2_ref[...] += jnp.sum(y2, axis=1, keepdims=True)
        q2_ref[...] += jnp.sum(y2 * y2, axis=1, keepdims=True)


def _stage34_kernel(y2_ref, s2_ref, q2_ref, g2_ref, be2_ref,
                    w3_ref, g3_ref, be3_ref,
                    o_ref,
                    hb_ref, m2_ref, sh_ref, a2_ref, c2_ref, a3_ref, c3_ref,
                    *, N, H, W):
    i = pl.program_id(0)
    inv_m = 1.0 / (N * H * W)

    @pl.when(i == 0)
    def _coef2():
        a2, c2 = _coeffs(g2_ref[...], be2_ref[...], s2_ref[...], q2_ref[...],
                         inv_m)
        a2_ref[...] = a2
        c2_ref[...] = c2
        m2_ref[...] = jnp.zeros_like(m2_ref)
        sh_ref[...] = jnp.zeros_like(sh_ref)

    @pl.when(i < N)
    def _phase_a():
        y2 = y2_ref[0].astype(jnp.float32)
        h = jnp.maximum(a2_ref[...] * y2 + c2_ref[...], 0.0)
        hb = h.astype(jnp.bfloat16)
        hb_ref[i] = hb
        m2_ref[...] += lax.dot_general(hb, hb, (((1,), (1,)), ((), ())),
                                       preferred_element_type=jnp.float32)
        sh_ref[...] += jnp.sum(hb.astype(jnp.float32), axis=1, keepdims=True)

    @pl.when(i == N)
    def _coef3():
        w3 = w3_ref[...]
        s3 = jnp.dot(w3, sh_ref[...], precision=_HI,
                     preferred_element_type=jnp.float32)
        a = jnp.dot(w3, m2_ref[...], precision=_HI,
                    preferred_element_type=jnp.float32)
        q3 = jnp.sum(a * w3, axis=1, keepdims=True)
        a3, c3 = _coeffs(g3_ref[...], be3_ref[...], s3, q3, inv_m)
        a3_ref[...] = a3
        c3_ref[...] = c3

    @pl.when(i >= N)
    def _phase_b():
        j = i - N
        w3b = w3_ref[...].astype(jnp.bfloat16)
        y3 = jnp.dot(w3b, hb_ref[j], preferred_element_type=jnp.float32)
        o_ref[0] = a3_ref[...] * y3 + c3_ref[...]


def kernel(x, w1_mat, w2_shift, w3_mat, b2, g1, be1, g2, be2, g3, be3):
    N, Cin, H, W = x.shape
    Cint = w1_mat.shape[0]
    Cout = w3_mat.shape[0]
    HW = H * W

    xr = x.reshape(N, Cin, HW)
    col = lambda c: pl.BlockSpec((c, 1), lambda i: (0, 0))
    arb = pltpu.CompilerParams(dimension_semantics=("arbitrary",))

    y2b, s2, q2 = pl.pallas_call(
        functools.partial(_stage12_kernel, N=N, H=H, W=W),
        out_shape=(jax.ShapeDtypeStruct((N, Cint, HW), jnp.bfloat16),
                   jax.ShapeDtypeStruct((Cint, 1), jnp.float32),
                   jax.ShapeDtypeStruct((Cint, 1), jnp.float32)),
        grid=(2 * N,),
        in_specs=[pl.BlockSpec((1, Cin, HW),
                               lambda i: (jnp.where(i < N, i, 0), 0, 0)),
                  pl.BlockSpec((Cint, Cin), lambda i: (0, 0)),
                  pl.BlockSpec((9, Cint, Cint), lambda i: (0, 0, 0)),
                  col(Cint), col(Cint), col(Cint)],
        out_specs=(pl.BlockSpec((1, Cint, HW),
                                lambda i: (jnp.where(i < N, 0, i - N), 0, 0)),
                   col(Cint), col(Cint)),
        scratch_shapes=[pltpu.VMEM((N, Cin, HW), jnp.bfloat16),
                        pltpu.VMEM((Cin, Cin), jnp.float32),
                        pltpu.VMEM((Cin, 1), jnp.float32),
                        pltpu.VMEM((Cint, 1), jnp.float32),
                        pltpu.VMEM((Cint, 1), jnp.float32)],
        compiler_params=arb,
    )(xr, w1_mat, w2_shift, b2, g1, be1)

    out = pl.pallas_call(
        functools.partial(_stage34_kernel, N=N, H=H, W=W),
        out_shape=jax.ShapeDtypeStruct((N, Cout, HW), jnp.float32),
        grid=(2 * N,),
        in_specs=[pl.BlockSpec((1, Cint, HW),
                               lambda i: (jnp.where(i < N, i, 0), 0, 0)),
                  col(Cint), col(Cint), col(Cint), col(Cint),
                  pl.BlockSpec((Cout, Cint), lambda i: (0, 0)),
                  col(Cout), col(Cout)],
        out_specs=pl.BlockSpec((1, Cout, HW),
                               lambda i: (jnp.where(i < N, 0, i - N), 0, 0)),
        scratch_shapes=[pltpu.VMEM((N, Cint, HW), jnp.bfloat16),
                        pltpu.VMEM((Cint, Cint), jnp.float32),
                        pltpu.VMEM((Cint, 1), jnp.float32),
                        pltpu.VMEM((Cint, 1), jnp.float32),
                        pltpu.VMEM((Cint, 1), jnp.float32),
                        pltpu.VMEM((Cout, 1), jnp.float32),
                        pltpu.VMEM((Cout, 1), jnp.float32)],
        compiler_params=arb,
    )(y2b, s2, q2, g2, be2, w3_mat, g3, be3)

    return out.reshape(N, Cout, H, W)


# speedup vs baseline: 1.0183x; 1.0183x over previous
import functools

import jax
import jax.numpy as jnp
from jax import lax
from jax.experimental import pallas as pl
from jax.experimental.pallas import tpu as pltpu

_EPS = 1e-3
_HI = lax.Precision.HIGHEST


def _coeffs(g, be, s, q, inv_m):
    mean = s * inv_m
    var = q * inv_m - mean * mean
    a = g * lax.rsqrt(var + _EPS)
    return a, be - mean * a


def _conv3x3(hb, w2b, H, W):
    hw = H * W
    idx = lax.broadcasted_iota(jnp.int32, (1, hw), 1)
    row = idx // W
    col = idx % W
    dxm = jnp.where(col >= 1, pltpu.roll(hb, shift=1, axis=1), 0)
    dxp = jnp.where(col <= W - 2, pltpu.roll(hb, shift=hw - 1, axis=1), 0)
    stack = jnp.concatenate([dxm, hb, dxp], axis=0)

    def wrow(dy):
        k = (dy + 1) * 3
        return jnp.concatenate([w2b[k], w2b[k + 1], w2b[k + 2]], axis=1)

    acc = jnp.dot(wrow(0), stack, preferred_element_type=jnp.float32)
    up = jnp.dot(wrow(-1), stack, preferred_element_type=jnp.float32)
    acc += jnp.where(row >= 1, pltpu.roll(up, shift=W, axis=1), 0.0)
    dn = jnp.dot(wrow(1), stack, preferred_element_type=jnp.float32)
    acc += jnp.where(row <= H - 2, pltpu.roll(dn, shift=hw - W, axis=1), 0.0)
    return acc


def _block_kernel(x_ref, w1_ref, w2_ref, b2_ref, g1_ref, be1_ref,
                  g2_ref, be2_ref, w3_ref, g3_ref, be3_ref,
                  o_ref,
                  buf_ref, m_ref, sx_ref, a1_ref, c1_ref, s2_ref, q2_ref,
                  m2_ref, sh_ref, a2_ref, c2_ref, a3_ref, c3_ref,
                  *, N, H, W, Cin, Cint):
    i = pl.program_id(0)
    inv_m = 1.0 / (N * H * W)

    @pl.when(i < N)
    def _phase_a():
        x = x_ref[0]
        xb = x.astype(jnp.bfloat16)
        buf_ref[i, 0:Cin, :] = xb

        @pl.when(i == 0)
        def _():
            m_ref[...] = jnp.zeros_like(m_ref)
            sx_ref[...] = jnp.zeros_like(sx_ref)

        m_ref[...] += lax.dot_general(xb, xb, (((1,), (1,)), ((), ())),
                                      preferred_element_type=jnp.float32)
        sx_ref[...] += jnp.sum(x, axis=1, keepdims=True)

    @pl.when(i == N)
    def _coef1():
        w1 = w1_ref[...]
        s1 = jnp.dot(w1, sx_ref[...], precision=_HI,
                     preferred_element_type=jnp.float32)
        a = jnp.dot(w1, m_ref[...], precision=_HI,
                    preferred_element_type=jnp.float32)
        q1 = jnp.sum(a * w1, axis=1, keepdims=True)
        a1, c1 = _coeffs(g1_ref[...], be1_ref[...], s1, q1, inv_m)
        a1_ref[...] = a1
        c1_ref[...] = c1
        s2_ref[...] = jnp.zeros_like(s2_ref)
        q2_ref[...] = jnp.zeros_like(q2_ref)

    @pl.when(jnp.logical_and(i >= N, i < 2 * N))
    def _phase_b():
        j = i - N
        xb = buf_ref[j, 0:Cin, :]
        w1b = w1_ref[...].astype(jnp.bfloat16)
        y1 = jnp.dot(w1b, xb, preferred_element_type=jnp.float32)
        h = jnp.maximum(a1_ref[...] * y1 + c1_ref[...], 0.0)
        w2b = w2_ref[...].astype(jnp.bfloat16)
        y2 = _conv3x3(h.astype(jnp.bfloat16), w2b, H, W) + b2_ref[...]
        buf_ref[j, 0:Cint, :] = y2.astype(jnp.bfloat16)
        s2_ref[...] += jnp.sum(y2, axis=1, keepdims=True)
        q2_ref[...] += jnp.sum(y2 * y2, axis=1, keepdims=True)

    @pl.when(i == 2 * N)
    def _coef2():
        a2, c2 = _coeffs(g2_ref[...], be2_ref[...], s2_ref[...], q2_ref[...],
                         inv_m)
        a2_ref[...] = a2
        c2_ref[...] = c2
        m2_ref[...] = jnp.zeros_like(m2_ref)
        sh_ref[...] = jnp.zeros_like(sh_ref)

    @pl.when(jnp.logical_and(i >= 2 * N, i < 3 * N))
    def _phase_c():
        j = i - 2 * N
        y2 = buf_ref[j, 0:Cint, :].astype(jnp.float32)
        h = jnp.maximum(a2_ref[...] * y2 + c2_ref[...], 0.0)
        hb = h.astype(jnp.bfloat16)
        buf_ref[j, 0:Cint, :] = hb
        m2_ref[...] += lax.dot_general(hb, hb, (((1,), (1,)), ((), ())),
                                       preferred_element_type=jnp.float32)
        sh_ref[...] += jnp.sum(hb.astype(jnp.float32), axis=1, keepdims=True)

    @pl.when(i == 3 * N)
    def _coef3():
        w3 = w3_ref[...]
        s3 = jnp.dot(w3, sh_ref[...], precision=_HI,
                     preferred_element_type=jnp.float32)
        a = jnp.dot(w3, m2_ref[...], precision=_HI,
                    preferred_element_type=jnp.float32)
        q3 = jnp.sum(a * w3, axis=1, keepdims=True)
        a3, c3 = _coeffs(g3_ref[...], be3_ref[...], s3, q3, inv_m)
        a3_ref[...] = a3
        c3_ref[...] = c3

    @pl.when(i >= 3 * N)
    def _phase_d():
        j = i - 3 * N
        hb = buf_ref[j, 0:Cint, :]
        w3b = w3_ref[...].astype(jnp.bfloat16)
        y3 = jnp.dot(w3b, hb, preferred_element_type=jnp.float32)
        o_ref[0] = a3_ref[...] * y3 + c3_ref[...]


def kernel(x, w1_mat, w2_shift, w3_mat, b2, g1, be1, g2, be2, g3, be3):
    N, Cin, H, W = x.shape
    Cint = w1_mat.shape[0]
    Cout = w3_mat.shape[0]
    HW = H * W
    C = max(Cin, Cint)

    xr = x.reshape(N, Cin, HW)
    col = lambda c: pl.BlockSpec((c, 1), lambda i: (0, 0))
    arb = pltpu.CompilerParams(dimension_semantics=("arbitrary",))

    out = pl.pallas_call(
        functools.partial(_block_kernel, N=N, H=H, W=W, Cin=Cin, Cint=Cint),
        out_shape=jax.ShapeDtypeStruct((N, Cout, HW), jnp.float32),
        grid=(4 * N,),
        in_specs=[pl.BlockSpec((1, Cin, HW),
                               lambda i: (jnp.where(i < N, i, 0), 0, 0)),
                  pl.BlockSpec((Cint, Cin), lambda i: (0, 0)),
                  pl.BlockSpec((9, Cint, Cint), lambda i: (0, 0, 0)),
                  col(Cint), col(Cint), col(Cint), col(Cint), col(Cint),
                  pl.BlockSpec((Cout, Cint), lambda i: (0, 0)),
                  col(Cout), col(Cout)],
        out_specs=pl.BlockSpec(
            (1, Cout, HW),
            lambda i: (jnp.where(i < 3 * N, 0, i - 3 * N), 0, 0)),
        scratch_shapes=[pltpu.VMEM((N, C, HW), jnp.bfloat16),
                        pltpu.VMEM((Cin, Cin), jnp.float32),
                        pltpu.VMEM((Cin, 1), jnp.float32),
                        pltpu.VMEM((Cint, 1), jnp.float32),
                        pltpu.VMEM((Cint, 1), jnp.float32),
                        pltpu.VMEM((Cint, 1), jnp.float32),
                        pltpu.VMEM((Cint, 1), jnp.float32),
                        pltpu.VMEM((Cint, Cint), jnp.float32),
                        pltpu.VMEM((Cint, 1), jnp.float32),
                        pltpu.VMEM((Cint, 1), jnp.float32),
                        pltpu.VMEM((Cint, 1), jnp.float32),
                        pltpu.VMEM((Cout, 1), jnp.float32),
                        pltpu.VMEM((Cout, 1), jnp.float32)],
        compiler_params=arb,
    )(xr, w1_mat, w2_shift, b2, g1, be1, g2, be2, w3_mat, g3, be3)

    return out.reshape(N, Cout, H, W)
